# baseline (speedup 1.0000x reference)
import functools

import jax
import jax.numpy as jnp
from jax import lax
from jax.experimental import pallas as pl
from jax.experimental.pallas import tpu as pltpu
from jax.experimental.pallas import tpu_sc as plsc

VOCAB = 100000
EMBED = 64
HIDDEN = 256
NUM_CLASSES = 10
PAD = VOCAB - 1
BATCH = 1024
SEQ = 200
N = BATCH * SEQ

NW = 32
CHUNK = 128
NSPLIT = 2
NT = N // NSPLIT

BT = 1024
BP = BT // 2


def _sc_gather(idx_w, idx_b, idx_t, emb_word, emb_bigram, emb_trigram, tok0):
    mesh = plsc.VectorSubcoreMesh(core_axis_name="c", subcore_axis_name="s")
    row_ty = jax.ShapeDtypeStruct((NT, EMBED), jnp.float32)
    per_w = NT // NW
    n_chunks = per_w // CHUNK
    assert per_w % CHUNK == 0

    @functools.partial(
        pl.kernel,
        mesh=mesh,
        out_type=(row_ty, row_ty, row_ty),
        compiler_params=pltpu.CompilerParams(
            use_tc_tiling_on_sc=False, needs_layout_passes=False),
        scratch_types=[
            pltpu.VMEM((CHUNK,), jnp.int32),
            pltpu.VMEM((CHUNK,), jnp.int32),
            pltpu.VMEM((CHUNK, EMBED), jnp.float32),
            pltpu.VMEM((CHUNK, EMBED), jnp.float32),
            pltpu.SemaphoreType.DMA,
            pltpu.SemaphoreType.DMA,
        ],
    )
    def gather_kernel(iw, ib, it, tw, tb, tt, ow, ob, ot,
                      idx0, idx1, rows0, rows1, sem0, sem1):
        info = plsc.get_sparse_core_info()
        nc = info.num_cores
        wid = lax.axis_index("s") * nc + lax.axis_index("c")
        src_base = tok0 + wid * per_w
        dst_base = wid * per_w
        idx_bufs = (idx0, idx1)
        row_bufs = (rows0, rows1)
        sems = (sem0, sem1)
        lane = lax.iota(jnp.int32, 16)

        def fixup(idxb, rowsb):
            def group(g, _):
                iv = idxb[pl.ds(g * 16, 16)]
                hit = (iv == PAD)
                any_hit = lax.reduce_max(hit.astype(jnp.int32), axes=(0,))

                @pl.when(any_hit > 0)
                def _():
                    rowv = g * 16 + lane

                    def word(w, _):
                        colv = jnp.zeros((16,), jnp.int32) + w
                        plsc.store_scatter(rowsb, [rowv, colv],
                                           jnp.zeros((16,), jnp.float32),
                                           mask=hit)
                        return 0
                    lax.fori_loop(0, EMBED, word, 0)
                return 0
            lax.fori_loop(0, CHUNK // 16, group, 0)

        for s, (ixs, tbl, out) in enumerate(
                ((iw, tw, ow), (ib, tb, ob), (it, tt, ot))):
            for b in range(2):
                pltpu.sync_copy(ixs.at[pl.ds(src_base + b * CHUNK, CHUNK)],
                                idx_bufs[b])
                pltpu.async_copy(tbl.at[idx_bufs[b]], row_bufs[b], sems[b])

            def pair(jj, _, ixs=ixs, tbl=tbl, out=out):
                for b in range(2):
                    j = jj * 2 + b
                    pltpu.make_async_copy(tbl.at[idx_bufs[b]], row_bufs[b],
                                          sems[b]).wait()
                    fixup(idx_bufs[b], row_bufs[b])
                    pltpu.sync_copy(row_bufs[b],
                                    out.at[pl.ds(dst_base + j * CHUNK, CHUNK),
                                           :])

                    @pl.when(j + 2 < n_chunks)
                    def _():
                        pltpu.sync_copy(
                            ixs.at[pl.ds(src_base + (j + 2) * CHUNK, CHUNK)],
                            idx_bufs[b])
                        pltpu.async_copy(tbl.at[idx_bufs[b]], row_bufs[b],
                                         sems[b])
                return 0

            lax.fori_loop(0, n_chunks // 2, pair, 0)
            if n_chunks % 2:
                j = n_chunks - 1
                pltpu.make_async_copy(tbl.at[idx_bufs[0]], row_bufs[0],
                                      sems[0]).wait()
                fixup(idx_bufs[0], row_bufs[0])
                pltpu.sync_copy(row_bufs[0],
                                out.at[pl.ds(dst_base + j * CHUNK, CHUNK), :])

    return gather_kernel(idx_w, idx_b, idx_t, emb_word, emb_bigram, emb_trigram)


def _mlp_kernel(gw_ref, gb_ref, gt_ref, w1_ref, b1_ref, w2_ref, b2_ref,
                ee_ref, eo_ref, o_ref):
    xcat = jnp.concatenate(
        [gw_ref[...], gb_ref[...], gt_ref[...]], axis=1
    ).astype(jnp.bfloat16)
    h = jnp.dot(xcat, w1_ref[...], preferred_element_type=jnp.float32)
    h = jnp.maximum(h + b1_ref[...], 0.0).astype(jnp.bfloat16)
    o = jnp.dot(h, w2_ref[...], preferred_element_type=jnp.float32)
    o = o + b2_ref[...]
    ot = jnp.transpose(o, (1, 0))
    ev = jnp.dot(ot[0:NUM_CLASSES, :].astype(jnp.bfloat16), ee_ref[...],
                 preferred_element_type=jnp.float32)
    od = jnp.dot(ot[16:16 + NUM_CLASSES, :].astype(jnp.bfloat16), eo_ref[...],
                 preferred_element_type=jnp.float32)
    o_ref[...] = ev + od


def _tc_mlp(gw, gb, gt, fc1_w, fc1_b, fc2_w, fc2_b):
    nb = NT // BT
    w1 = fc1_w.T.astype(jnp.bfloat16)
    z = jnp.zeros((EMBED, HIDDEN), dtype=jnp.bfloat16)
    wd = []
    for s in range(3):
        ws = w1[s * EMBED:(s + 1) * EMBED]
        wd.append(jnp.block([[ws, z], [z, ws]]))
    wcat = jnp.concatenate(wd, axis=0)
    w2 = fc2_w.T.astype(jnp.bfloat16)
    w2p = jnp.zeros((2 * HIDDEN, 32), dtype=jnp.bfloat16)
    w2p = w2p.at[:HIDDEN, :NUM_CLASSES].set(w2)
    w2p = w2p.at[HIDDEN:, 16:16 + NUM_CLASSES].set(w2)
    b1d = jnp.concatenate([fc1_b, fc1_b]).reshape(1, 2 * HIDDEN)
    b2p = jnp.zeros((1, 32), dtype=jnp.float32)
    b2p = b2p.at[0, :NUM_CLASSES].set(fc2_b)
    b2p = b2p.at[0, 16:16 + NUM_CLASSES].set(fc2_b)

    pr = lax.iota(jnp.int32, BP).reshape(BP, 1)
    qc = lax.iota(jnp.int32, BT).reshape(1, BT)
    ee = (qc == 2 * pr).astype(jnp.bfloat16)
    eo = (qc == 2 * pr + 1).astype(jnp.bfloat16)

    g_spec = pl.BlockSpec((BP, 2 * EMBED), lambda i: (i, 0))
    return pl.pallas_call(
        _mlp_kernel,
        grid=(nb,),
        in_specs=[
            g_spec, g_spec, g_spec,
            pl.BlockSpec((3 * 2 * EMBED, 2 * HIDDEN), lambda i: (0, 0)),
            pl.BlockSpec((1, 2 * HIDDEN), lambda i: (0, 0)),
            pl.BlockSpec((2 * HIDDEN, 32), lambda i: (0, 0)),
            pl.BlockSpec((1, 32), lambda i: (0, 0)),
            pl.BlockSpec((BP, BT), lambda i: (0, 0)),
            pl.BlockSpec((BP, BT), lambda i: (0, 0)),
        ],
        out_specs=pl.BlockSpec((NUM_CLASSES, BT), lambda i: (0, i)),
        out_shape=jax.ShapeDtypeStruct((NUM_CLASSES, NT), jnp.float32),
    )(gw, gb, gt, wcat, b1d, w2p, b2p, ee, eo)


def kernel(x, emb_word, emb_bigram, emb_trigram, fc1_w, fc1_b, fc2_w, fc2_b):
    xt = jnp.transpose(x, (0, 2, 1))
    iw = xt[0].reshape(N)
    ib = xt[1].reshape(N)
    it = xt[2].reshape(N)
    emb_word, emb_bigram, emb_trigram = (
        lax.optimization_barrier(t.reshape(-1)).reshape(VOCAB, EMBED)
        for t in (emb_word, emb_bigram, emb_trigram))
    outs = []
    for k in range(NSPLIT):
        gw, gb, gt = _sc_gather(iw, ib, it, emb_word, emb_bigram, emb_trigram,
                                k * NT)
        gw2 = gw.reshape(NT // 2, 2 * EMBED)
        gb2 = gb.reshape(NT // 2, 2 * EMBED)
        gt2 = gt.reshape(NT // 2, 2 * EMBED)
        outs.append(_tc_mlp(gw2, gb2, gt2, fc1_w, fc1_b, fc2_w, fc2_b))
    o_np = jnp.concatenate(outs, axis=1)
    return o_np.reshape(NUM_CLASSES, SEQ, BATCH).transpose(2, 1, 0)


# speedup vs baseline: 1.0727x; 1.0727x over previous
import functools

import jax
import jax.numpy as jnp
from jax import lax
from jax.experimental import pallas as pl
from jax.experimental.pallas import tpu as pltpu
from jax.experimental.pallas import tpu_sc as plsc

VOCAB = 100000
EMBED = 64
HIDDEN = 256
NUM_CLASSES = 10
PAD = VOCAB - 1
BATCH = 1024
SEQ = 200
N = BATCH * SEQ

NW = 32
NSPLIT = 2
NT = N // NSPLIT
NPAIR = NT // 2
CP = 64

BT = 1024
BP = BT // 2


def _sc_gather(idx_w, idx_b, idx_t, emb_word, emb_bigram, emb_trigram, tok0):
    mesh = plsc.VectorSubcoreMesh(core_axis_name="c", subcore_axis_name="s")
    row_ty = jax.ShapeDtypeStruct((NPAIR, 2 * EMBED), jnp.float32)
    per_w = NPAIR // NW
    n_chunks = per_w // CP
    assert per_w % CP == 0

    @functools.partial(
        pl.kernel,
        mesh=mesh,
        out_type=(row_ty, row_ty, row_ty),
        compiler_params=pltpu.CompilerParams(
            use_tc_tiling_on_sc=False, needs_layout_passes=False),
        scratch_types=[
            pltpu.VMEM((2 * CP,), jnp.int32),
            pltpu.VMEM((2 * CP,), jnp.int32),
            pltpu.VMEM((2 * CP, EMBED), jnp.float32),
            pltpu.VMEM((2 * CP, EMBED), jnp.float32),
            pltpu.SemaphoreType.DMA,
            pltpu.SemaphoreType.DMA,
        ],
    )
    def gather_kernel(iw, ib, it, tw, tb, tt, ow, ob, ot,
                      idx0, idx1, rows0, rows1, sem0, sem1):
        info = plsc.get_sparse_core_info()
        nc = info.num_cores
        wid = lax.axis_index("s") * nc + lax.axis_index("c")
        lo_base = tok0 + wid * per_w
        hi_base = lo_base + NT // 2
        dst_base = wid * per_w
        idx_bufs = (idx0, idx1)
        row_bufs = (rows0, rows1)
        sems = (sem0, sem1)
        lane = lax.iota(jnp.int32, 16)

        def fixup(idxb, rowsb):
            def group(g, _):
                iv = idxb[pl.ds(g * 16, 16)]
                hit = (iv == PAD)
                any_hit = lax.reduce_max(hit.astype(jnp.int32), axes=(0,))

                @pl.when(any_hit > 0)
                def _():
                    rowv = g * 16 + lane

                    def word(w, _):
                        colv = jnp.zeros((16,), jnp.int32) + w
                        plsc.store_scatter(rowsb, [rowv, colv],
                                           jnp.zeros((16,), jnp.float32),
                                           mask=hit)
                        return 0
                    lax.fori_loop(0, EMBED, word, 0)
                return 0
            lax.fori_loop(0, (2 * CP) // 16, group, 0)

        def load_and_fire(b, c, ixs, tbl):
            pltpu.sync_copy(ixs.at[pl.ds(lo_base + c * CP, CP)],
                            idx_bufs[b].at[pl.ds(0, CP)])
            pltpu.sync_copy(ixs.at[pl.ds(hi_base + c * CP, CP)],
                            idx_bufs[b].at[pl.ds(CP, CP)])
            pltpu.async_copy(tbl.at[idx_bufs[b]], row_bufs[b], sems[b])

        def drain(b, c, tbl, out):
            pltpu.make_async_copy(tbl.at[idx_bufs[b]], row_bufs[b],
                                  sems[b]).wait()
            fixup(idx_bufs[b], row_bufs[b])
            r0 = dst_base + c * CP
            pltpu.sync_copy(row_bufs[b].at[pl.ds(0, CP), :],
                            out.at[pl.ds(r0, CP), pl.ds(0, EMBED)])
            pltpu.sync_copy(row_bufs[b].at[pl.ds(CP, CP), :],
                            out.at[pl.ds(r0, CP), pl.ds(EMBED, EMBED)])

        for s, (ixs, tbl, out) in enumerate(
                ((iw, tw, ow), (ib, tb, ob), (it, tt, ot))):
            for b in range(2):
                load_and_fire(b, b, ixs, tbl)

            def pair(jj, _, ixs=ixs, tbl=tbl, out=out):
                for b in range(2):
                    j = jj * 2 + b
                    drain(b, j, tbl, out)

                    @pl.when(j + 2 < n_chunks)
                    def _():
                        load_and_fire(b, j + 2, ixs, tbl)
                return 0

            lax.fori_loop(0, n_chunks // 2, pair, 0)
            if n_chunks % 2:
                drain(0, n_chunks - 1, tbl, out)

    return gather_kernel(idx_w, idx_b, idx_t, emb_word, emb_bigram, emb_trigram)


def _mlp_kernel(gw_ref, gb_ref, gt_ref, w1_ref, b1_ref, w2_ref, b2_ref,
                olo_ref, ohi_ref):
    xcat = jnp.concatenate(
        [gw_ref[...], gb_ref[...], gt_ref[...]], axis=1
    ).astype(jnp.bfloat16)
    h = jnp.dot(xcat, w1_ref[...], preferred_element_type=jnp.float32)
    h = jnp.maximum(h + b1_ref[...], 0.0).astype(jnp.bfloat16)
    o = jnp.dot(h, w2_ref[...], preferred_element_type=jnp.float32)
    o = o + b2_ref[...]
    ot = jnp.transpose(o, (1, 0))
    olo_ref[...] = ot[0:NUM_CLASSES, :]
    ohi_ref[...] = ot[16:16 + NUM_CLASSES, :]


def _tc_mlp(gw, gb, gt, fc1_w, fc1_b, fc2_w, fc2_b):
    nb = NPAIR // BP
    w1 = fc1_w.T.astype(jnp.bfloat16)
    z = jnp.zeros((EMBED, HIDDEN), dtype=jnp.bfloat16)
    wd = []
    for s in range(3):
        ws = w1[s * EMBED:(s + 1) * EMBED]
        wd.append(jnp.block([[ws, z], [z, ws]]))
    wcat = jnp.concatenate(wd, axis=0)
    w2 = fc2_w.T.astype(jnp.bfloat16)
    w2p = jnp.zeros((2 * HIDDEN, 32), dtype=jnp.bfloat16)
    w2p = w2p.at[:HIDDEN, :NUM_CLASSES].set(w2)
    w2p = w2p.at[HIDDEN:, 16:16 + NUM_CLASSES].set(w2)
    b1d = jnp.concatenate([fc1_b, fc1_b]).reshape(1, 2 * HIDDEN)
    b2p = jnp.zeros((1, 32), dtype=jnp.float32)
    b2p = b2p.at[0, :NUM_CLASSES].set(fc2_b)
    b2p = b2p.at[0, 16:16 + NUM_CLASSES].set(fc2_b)

    g_spec = pl.BlockSpec((BP, 2 * EMBED), lambda i: (i, 0))
    o_spec = pl.BlockSpec((NUM_CLASSES, BP), lambda i: (0, i))
    o_ty = jax.ShapeDtypeStruct((NUM_CLASSES, NPAIR), jnp.float32)
    return pl.pallas_call(
        _mlp_kernel,
        grid=(nb,),
        in_specs=[
            g_spec, g_spec, g_spec,
            pl.BlockSpec((3 * 2 * EMBED, 2 * HIDDEN), lambda i: (0, 0)),
            pl.BlockSpec((1, 2 * HIDDEN), lambda i: (0, 0)),
            pl.BlockSpec((2 * HIDDEN, 32), lambda i: (0, 0)),
            pl.BlockSpec((1, 32), lambda i: (0, 0)),
        ],
        out_specs=(o_spec, o_spec),
        out_shape=(o_ty, o_ty),
    )(gw, gb, gt, wcat, b1d, w2p, b2p)


def kernel(x, emb_word, emb_bigram, emb_trigram, fc1_w, fc1_b, fc2_w, fc2_b):
    xt = jnp.transpose(x, (0, 2, 1))
    iw = xt[0].reshape(N)
    ib = xt[1].reshape(N)
    it = xt[2].reshape(N)
    emb_word, emb_bigram, emb_trigram = (
        lax.optimization_barrier(t.reshape(-1)).reshape(VOCAB, EMBED)
        for t in (emb_word, emb_bigram, emb_trigram))
    panels = []
    for k in range(NSPLIT):
        gw, gb, gt = _sc_gather(iw, ib, it, emb_word, emb_bigram, emb_trigram,
                                k * NT)
        olo, ohi = _tc_mlp(gw, gb, gt, fc1_w, fc1_b, fc2_w, fc2_b)
        panels += [olo, ohi]
    o_np = jnp.concatenate(panels, axis=1)
    return o_np.reshape(NUM_CLASSES, SEQ, BATCH).transpose(2, 1, 0)


# speedup vs baseline: 1.1659x; 1.0869x over previous
import functools

import jax
import jax.numpy as jnp
from jax import lax
from jax.experimental import pallas as pl
from jax.experimental.pallas import tpu as pltpu
from jax.experimental.pallas import tpu_sc as plsc

VOCAB = 100000
EMBED = 64
HIDDEN = 256
NUM_CLASSES = 10
PAD = VOCAB - 1
BATCH = 1024
SEQ = 200
N = BATCH * SEQ

NW = 32
NSPLIT = 2
NT = N // NSPLIT
NPAIR = NT // 2
CP = 64

BT = 1024
BP = BT // 2


def _sc_gather(idx_vec, table, tok0):
    mesh = plsc.VectorSubcoreMesh(core_axis_name="c", subcore_axis_name="s")
    row_ty = jax.ShapeDtypeStruct((NPAIR, 2 * EMBED), jnp.float32)
    per_w = NPAIR // NW
    n_chunks = per_w // CP
    assert per_w % CP == 0

    @functools.partial(
        pl.kernel,
        mesh=mesh,
        out_type=row_ty,
        compiler_params=pltpu.CompilerParams(
            use_tc_tiling_on_sc=False, needs_layout_passes=False),
        scratch_types=[
            pltpu.VMEM((2 * CP,), jnp.int32),
            pltpu.VMEM((2 * CP,), jnp.int32),
            pltpu.VMEM((2 * CP, EMBED), jnp.float32),
            pltpu.VMEM((2 * CP, EMBED), jnp.float32),
            pltpu.SemaphoreType.DMA,
            pltpu.SemaphoreType.DMA,
        ],
    )
    def gather_kernel(ixs, tbl, out,
                      idx0, idx1, rows0, rows1, sem0, sem1):
        info = plsc.get_sparse_core_info()
        nc = info.num_cores
        wid = lax.axis_index("s") * nc + lax.axis_index("c")
        lo_base = tok0 + wid * per_w
        hi_base = lo_base + NT // 2
        dst_base = wid * per_w
        idx_bufs = (idx0, idx1)
        row_bufs = (rows0, rows1)
        sems = (sem0, sem1)
        lane = lax.iota(jnp.int32, 16)

        def fixup(idxb, rowsb):
            def group(g, _):
                iv = idxb[pl.ds(g * 16, 16)]
                hit = (iv == PAD)
                any_hit = lax.reduce_max(hit.astype(jnp.int32), axes=(0,))

                @pl.when(any_hit > 0)
                def _():
                    rowv = g * 16 + lane

                    def word(w, _):
                        colv = jnp.zeros((16,), jnp.int32) + w
                        plsc.store_scatter(rowsb, [rowv, colv],
                                           jnp.zeros((16,), jnp.float32),
                                           mask=hit)
                        return 0
                    lax.fori_loop(0, EMBED, word, 0)
                return 0
            lax.fori_loop(0, (2 * CP) // 16, group, 0)

        def load_and_fire(b, c):
            pltpu.sync_copy(ixs.at[pl.ds(lo_base + c * CP, CP)],
                            idx_bufs[b].at[pl.ds(0, CP)])
            pltpu.sync_copy(ixs.at[pl.ds(hi_base + c * CP, CP)],
                            idx_bufs[b].at[pl.ds(CP, CP)])
            pltpu.async_copy(tbl.at[idx_bufs[b]], row_bufs[b], sems[b])

        def drain(b, c):
            pltpu.make_async_copy(tbl.at[idx_bufs[b]], row_bufs[b],
                                  sems[b]).wait()
            fixup(idx_bufs[b], row_bufs[b])
            r0 = dst_base + c * CP
            pltpu.sync_copy(row_bufs[b].at[pl.ds(0, CP), :],
                            out.at[pl.ds(r0, CP), pl.ds(0, EMBED)])
            pltpu.sync_copy(row_bufs[b].at[pl.ds(CP, CP), :],
                            out.at[pl.ds(r0, CP), pl.ds(EMBED, EMBED)])

        for b in range(2):
            load_and_fire(b, b)

        def pair(jj, _):
            for b in range(2):
                j = jj * 2 + b
                drain(b, j)

                @pl.when(j + 2 < n_chunks)
                def _():
                    load_and_fire(b, j + 2)
            return 0

        lax.fori_loop(0, n_chunks // 2, pair, 0)
        if n_chunks % 2:
            drain(0, n_chunks - 1)

    return gather_kernel(idx_vec, table)


def _mlp_kernel(gw_ref, gb_ref, gt_ref, w1_ref, b1_ref, w2_ref, b2_ref,
                olo_ref, ohi_ref):
    xcat = jnp.concatenate(
        [gw_ref[...], gb_ref[...], gt_ref[...]], axis=1
    ).astype(jnp.bfloat16)
    h = jnp.dot(xcat, w1_ref[...], preferred_element_type=jnp.float32)
    h = jnp.maximum(h + b1_ref[...], 0.0).astype(jnp.bfloat16)
    o = jnp.dot(h, w2_ref[...], preferred_element_type=jnp.float32)
    o = o + b2_ref[...]
    ot = jnp.transpose(o, (1, 0))
    olo_ref[...] = ot[0:NUM_CLASSES, :]
    ohi_ref[...] = ot[16:16 + NUM_CLASSES, :]


def _tc_mlp(gw, gb, gt, fc1_w, fc1_b, fc2_w, fc2_b):
    nb = NPAIR // BP
    w1 = fc1_w.T.astype(jnp.bfloat16)
    z = jnp.zeros((EMBED, HIDDEN), dtype=jnp.bfloat16)
    wd = []
    for s in range(3):
        ws = w1[s * EMBED:(s + 1) * EMBED]
        wd.append(jnp.block([[ws, z], [z, ws]]))
    wcat = jnp.concatenate(wd, axis=0)
    w2 = fc2_w.T.astype(jnp.bfloat16)
    w2p = jnp.zeros((2 * HIDDEN, 32), dtype=jnp.bfloat16)
    w2p = w2p.at[:HIDDEN, :NUM_CLASSES].set(w2)
    w2p = w2p.at[HIDDEN:, 16:16 + NUM_CLASSES].set(w2)
    b1d = jnp.concatenate([fc1_b, fc1_b]).reshape(1, 2 * HIDDEN)
    b2p = jnp.zeros((1, 32), dtype=jnp.float32)
    b2p = b2p.at[0, :NUM_CLASSES].set(fc2_b)
    b2p = b2p.at[0, 16:16 + NUM_CLASSES].set(fc2_b)

    g_spec = pl.BlockSpec((BP, 2 * EMBED), lambda i: (i, 0))
    o_spec = pl.BlockSpec((NUM_CLASSES, BP), lambda i: (0, i))
    o_ty = jax.ShapeDtypeStruct((NUM_CLASSES, NPAIR), jnp.float32)
    return pl.pallas_call(
        _mlp_kernel,
        grid=(nb,),
        in_specs=[
            g_spec, g_spec, g_spec,
            pl.BlockSpec((3 * 2 * EMBED, 2 * HIDDEN), lambda i: (0, 0)),
            pl.BlockSpec((1, 2 * HIDDEN), lambda i: (0, 0)),
            pl.BlockSpec((2 * HIDDEN, 32), lambda i: (0, 0)),
            pl.BlockSpec((1, 32), lambda i: (0, 0)),
        ],
        out_specs=(o_spec, o_spec),
        out_shape=(o_ty, o_ty),
    )(gw, gb, gt, wcat, b1d, w2p, b2p)


def kernel(x, emb_word, emb_bigram, emb_trigram, fc1_w, fc1_b, fc2_w, fc2_b):
    xt = jnp.transpose(x, (0, 2, 1))
    iw = xt[0].reshape(N)
    ib = xt[1].reshape(N)
    it = xt[2].reshape(N)
    emb_word, emb_bigram, emb_trigram = (
        lax.optimization_barrier(t.reshape(-1)).reshape(VOCAB, EMBED)
        for t in (emb_word, emb_bigram, emb_trigram))
    panels = []
    for k in range(NSPLIT):
        gw = _sc_gather(iw, emb_word, k * NT)
        gb = _sc_gather(ib, emb_bigram, k * NT)
        gt = _sc_gather(it, emb_trigram, k * NT)
        olo, ohi = _tc_mlp(gw, gb, gt, fc1_w, fc1_b, fc2_w, fc2_b)
        panels += [olo, ohi]
    o_np = jnp.concatenate(panels, axis=1)
    return o_np.reshape(NUM_CLASSES, SEQ, BATCH).transpose(2, 1, 0)


# speedup vs baseline: 1.2436x; 1.0666x over previous
import functools

import jax
import jax.numpy as jnp
from jax import lax
from jax.experimental import pallas as pl
from jax.experimental.pallas import tpu as pltpu
from jax.experimental.pallas import tpu_sc as plsc

VOCAB = 100000
EMBED = 64
HIDDEN = 256
NUM_CLASSES = 10
PAD = VOCAB - 1
BATCH = 1024
SEQ = 200
N = BATCH * SEQ

NW = 32
NSPLIT = 2
NT = N // NSPLIT
NPAIR = NT // 2
CP = 64

BT = 2048
BP = BT // 2


def _sc_gather(idx_vec, table, tok0):
    mesh = plsc.VectorSubcoreMesh(core_axis_name="c", subcore_axis_name="s")
    row_ty = jax.ShapeDtypeStruct((NPAIR, 2 * EMBED), jnp.float32)
    per_w = NPAIR // NW
    n_chunks = per_w // CP
    assert per_w % CP == 0

    @functools.partial(
        pl.kernel,
        mesh=mesh,
        out_type=row_ty,
        compiler_params=pltpu.CompilerParams(
            use_tc_tiling_on_sc=False, needs_layout_passes=False),
        scratch_types=[
            pltpu.VMEM((2 * CP,), jnp.int32),
            pltpu.VMEM((2 * CP,), jnp.int32),
            pltpu.VMEM((2 * CP, EMBED), jnp.float32),
            pltpu.VMEM((2 * CP, EMBED), jnp.float32),
            pltpu.SemaphoreType.DMA,
            pltpu.SemaphoreType.DMA,
        ],
    )
    def gather_kernel(ixs, tbl, out,
                      idx0, idx1, rows0, rows1, sem0, sem1):
        info = plsc.get_sparse_core_info()
        nc = info.num_cores
        wid = lax.axis_index("s") * nc + lax.axis_index("c")
        lo_base = tok0 + wid * per_w
        hi_base = lo_base + NT // 2
        dst_base = wid * per_w
        idx_bufs = (idx0, idx1)
        row_bufs = (rows0, rows1)
        sems = (sem0, sem1)
        lane = lax.iota(jnp.int32, 16)

        def fixup(idxb, rowsb):
            def group(g, _):
                iv = idxb[pl.ds(g * 16, 16)]
                hit = (iv == PAD)
                any_hit = lax.reduce_max(hit.astype(jnp.int32), axes=(0,))

                @pl.when(any_hit > 0)
                def _():
                    rowv = g * 16 + lane

                    def word(w, _):
                        colv = jnp.zeros((16,), jnp.int32) + w
                        plsc.store_scatter(rowsb, [rowv, colv],
                                           jnp.zeros((16,), jnp.float32),
                                           mask=hit)
                        return 0
                    lax.fori_loop(0, EMBED, word, 0)
                return 0
            lax.fori_loop(0, (2 * CP) // 16, group, 0)

        def load_and_fire(b, c):
            pltpu.sync_copy(ixs.at[pl.ds(lo_base + c * CP, CP)],
                            idx_bufs[b].at[pl.ds(0, CP)])
            pltpu.sync_copy(ixs.at[pl.ds(hi_base + c * CP, CP)],
                            idx_bufs[b].at[pl.ds(CP, CP)])
            pltpu.async_copy(tbl.at[idx_bufs[b]], row_bufs[b], sems[b])

        def drain(b, c):
            pltpu.make_async_copy(tbl.at[idx_bufs[b]], row_bufs[b],
                                  sems[b]).wait()
            fixup(idx_bufs[b], row_bufs[b])
            r0 = dst_base + c * CP
            pltpu.sync_copy(row_bufs[b].at[pl.ds(0, CP), :],
                            out.at[pl.ds(r0, CP), pl.ds(0, EMBED)])
            pltpu.sync_copy(row_bufs[b].at[pl.ds(CP, CP), :],
                            out.at[pl.ds(r0, CP), pl.ds(EMBED, EMBED)])

        for b in range(2):
            load_and_fire(b, b)

        def pair(jj, _):
            for b in range(2):
                j = jj * 2 + b
                drain(b, j)

                @pl.when(j + 2 < n_chunks)
                def _():
                    load_and_fire(b, j + 2)
            return 0

        lax.fori_loop(0, n_chunks // 2, pair, 0)
        if n_chunks % 2:
            drain(0, n_chunks - 1)

    return gather_kernel(idx_vec, table)


def _mlp_kernel(gw_ref, gb_ref, gt_ref, w1_ref, b1_ref, w2_ref, b2_ref,
                olo_ref, ohi_ref):
    xcat = jnp.concatenate(
        [gw_ref[...], gb_ref[...], gt_ref[...]], axis=1
    ).astype(jnp.bfloat16)
    h = jnp.dot(xcat, w1_ref[...], preferred_element_type=jnp.float32)
    h = jnp.maximum(h + b1_ref[...], 0.0).astype(jnp.bfloat16)
    o = jnp.dot(h, w2_ref[...], preferred_element_type=jnp.float32)
    o = o + b2_ref[...]
    ot = jnp.transpose(o, (1, 0))
    olo_ref[...] = ot[0:NUM_CLASSES, :]
    ohi_ref[...] = ot[16:16 + NUM_CLASSES, :]


def _tc_mlp(gw, gb, gt, fc1_w, fc1_b, fc2_w, fc2_b):
    nb = NPAIR // BP
    w1 = fc1_w.T.astype(jnp.bfloat16)
    z = jnp.zeros((EMBED, HIDDEN), dtype=jnp.bfloat16)
    wd = []
    for s in range(3):
        ws = w1[s * EMBED:(s + 1) * EMBED]
        wd.append(jnp.block([[ws, z], [z, ws]]))
    wcat = jnp.concatenate(wd, axis=0)
    w2 = fc2_w.T.astype(jnp.bfloat16)
    w2p = jnp.zeros((2 * HIDDEN, 32), dtype=jnp.bfloat16)
    w2p = w2p.at[:HIDDEN, :NUM_CLASSES].set(w2)
    w2p = w2p.at[HIDDEN:, 16:16 + NUM_CLASSES].set(w2)
    b1d = jnp.concatenate([fc1_b, fc1_b]).reshape(1, 2 * HIDDEN)
    b2p = jnp.zeros((1, 32), dtype=jnp.float32)
    b2p = b2p.at[0, :NUM_CLASSES].set(fc2_b)
    b2p = b2p.at[0, 16:16 + NUM_CLASSES].set(fc2_b)

    g_spec = pl.BlockSpec((BP, 2 * EMBED), lambda i: (i, 0))
    o_spec = pl.BlockSpec((NUM_CLASSES, BP), lambda i: (0, i))
    o_ty = jax.ShapeDtypeStruct((NUM_CLASSES, NPAIR), jnp.float32)
    return pl.pallas_call(
        _mlp_kernel,
        grid=(nb,),
        in_specs=[
            g_spec, g_spec, g_spec,
            pl.BlockSpec((3 * 2 * EMBED, 2 * HIDDEN), lambda i: (0, 0)),
            pl.BlockSpec((1, 2 * HIDDEN), lambda i: (0, 0)),
            pl.BlockSpec((2 * HIDDEN, 32), lambda i: (0, 0)),
            pl.BlockSpec((1, 32), lambda i: (0, 0)),
        ],
        out_specs=(o_spec, o_spec),
        out_shape=(o_ty, o_ty),
    )(gw, gb, gt, wcat, b1d, w2p, b2p)


def kernel(x, emb_word, emb_bigram, emb_trigram, fc1_w, fc1_b, fc2_w, fc2_b):
    xt = jnp.transpose(x, (0, 2, 1))
    iw = xt[0].reshape(N)
    ib = xt[1].reshape(N)
    it = xt[2].reshape(N)
    emb_word, emb_bigram, emb_trigram = (
        lax.optimization_barrier(t.reshape(-1)).reshape(VOCAB, EMBED)
        for t in (emb_word, emb_bigram, emb_trigram))
    panels = []
    for k in range(NSPLIT):
        gw = _sc_gather(iw, emb_word, k * NT)
        gb = _sc_gather(ib, emb_bigram, k * NT)
        gt = _sc_gather(it, emb_trigram, k * NT)
        olo, ohi = _tc_mlp(gw, gb, gt, fc1_w, fc1_b, fc2_w, fc2_b)
        panels += [olo, ohi]
    o_np = jnp.concatenate(panels, axis=1)
    return o_np.reshape(NUM_CLASSES, SEQ, BATCH).transpose(2, 1, 0)


# speedup vs baseline: 1.2752x; 1.0255x over previous
import functools

import jax
import jax.numpy as jnp
from jax import lax
from jax.experimental import pallas as pl
from jax.experimental.pallas import tpu as pltpu
from jax.experimental.pallas import tpu_sc as plsc

VOCAB = 100000
EMBED = 64
HIDDEN = 256
NUM_CLASSES = 10
PAD = VOCAB - 1
BATCH = 1024
SEQ = 200
N = BATCH * SEQ

NW = 32
NSPLIT = 2
NT = N // NSPLIT
NPAIR = NT // 2
CP = 64

BT = 4096
BP = BT // 2


def _sc_gather(idx_vec, table, tok0):
    mesh = plsc.VectorSubcoreMesh(core_axis_name="c", subcore_axis_name="s")
    row_ty = jax.ShapeDtypeStruct((NPAIR, 2 * EMBED), jnp.float32)
    per_w = NPAIR // NW
    n_chunks = per_w // CP
    assert per_w % CP == 0

    @functools.partial(
        pl.kernel,
        mesh=mesh,
        out_type=row_ty,
        compiler_params=pltpu.CompilerParams(
            use_tc_tiling_on_sc=False, needs_layout_passes=False),
        scratch_types=[
            pltpu.VMEM((2 * CP,), jnp.int32),
            pltpu.VMEM((2 * CP,), jnp.int32),
            pltpu.VMEM((2 * CP, EMBED), jnp.float32),
            pltpu.VMEM((2 * CP, EMBED), jnp.float32),
            pltpu.SemaphoreType.DMA,
            pltpu.SemaphoreType.DMA,
        ],
    )
    def gather_kernel(ixs, tbl, out,
                      idx0, idx1, rows0, rows1, sem0, sem1):
        info = plsc.get_sparse_core_info()
        nc = info.num_cores
        wid = lax.axis_index("s") * nc + lax.axis_index("c")
        lo_base = tok0 + wid * per_w
        hi_base = lo_base + NT // 2
        dst_base = wid * per_w
        idx_bufs = (idx0, idx1)
        row_bufs = (rows0, rows1)
        sems = (sem0, sem1)
        lane = lax.iota(jnp.int32, 16)

        def fixup(idxb, rowsb):
            def group(g, _):
                iv = idxb[pl.ds(g * 16, 16)]
                hit = (iv == PAD)
                any_hit = lax.reduce_max(hit.astype(jnp.int32), axes=(0,))

                @pl.when(any_hit > 0)
                def _():
                    rowv = g * 16 + lane

                    def word(w, _):
                        colv = jnp.zeros((16,), jnp.int32) + w
                        plsc.store_scatter(rowsb, [rowv, colv],
                                           jnp.zeros((16,), jnp.float32),
                                           mask=hit)
                        return 0
                    lax.fori_loop(0, EMBED, word, 0)
                return 0
            lax.fori_loop(0, (2 * CP) // 16, group, 0)

        def load_and_fire(b, c):
            pltpu.sync_copy(ixs.at[pl.ds(lo_base + c * CP, CP)],
                            idx_bufs[b].at[pl.ds(0, CP)])
            pltpu.sync_copy(ixs.at[pl.ds(hi_base + c * CP, CP)],
                            idx_bufs[b].at[pl.ds(CP, CP)])
            pltpu.async_copy(tbl.at[idx_bufs[b]], row_bufs[b], sems[b])

        def drain(b, c):
            pltpu.make_async_copy(tbl.at[idx_bufs[b]], row_bufs[b],
                                  sems[b]).wait()
            fixup(idx_bufs[b], row_bufs[b])
            r0 = dst_base + c * CP
            pltpu.sync_copy(row_bufs[b].at[pl.ds(0, CP), :],
                            out.at[pl.ds(r0, CP), pl.ds(0, EMBED)])
            pltpu.sync_copy(row_bufs[b].at[pl.ds(CP, CP), :],
                            out.at[pl.ds(r0, CP), pl.ds(EMBED, EMBED)])

        for b in range(2):
            load_and_fire(b, b)

        def pair(jj, _):
            for b in range(2):
                j = jj * 2 + b
                drain(b, j)

                @pl.when(j + 2 < n_chunks)
                def _():
                    load_and_fire(b, j + 2)
            return 0

        lax.fori_loop(0, n_chunks // 2, pair, 0)
        if n_chunks % 2:
            drain(0, n_chunks - 1)

    return gather_kernel(idx_vec, table)


def _mlp_kernel(gw_ref, gb_ref, gt_ref, w1_ref, b1_ref, w2_ref, b2_ref,
                olo_ref, ohi_ref):
    xcat = jnp.concatenate(
        [gw_ref[...], gb_ref[...], gt_ref[...]], axis=1
    ).astype(jnp.bfloat16)
    h = jnp.dot(xcat, w1_ref[...], preferred_element_type=jnp.float32)
    h = jnp.maximum(h + b1_ref[...], 0.0).astype(jnp.bfloat16)
    o = jnp.dot(h, w2_ref[...], preferred_element_type=jnp.float32)
    o = o + b2_ref[...]
    ot = jnp.transpose(o, (1, 0))
    olo_ref[...] = ot[0:NUM_CLASSES, :]
    ohi_ref[...] = ot[16:16 + NUM_CLASSES, :]


def _tc_mlp(gw, gb, gt, fc1_w, fc1_b, fc2_w, fc2_b):
    nb = NPAIR // BP
    w1 = fc1_w.T.astype(jnp.bfloat16)
    z = jnp.zeros((EMBED, HIDDEN), dtype=jnp.bfloat16)
    wd = []
    for s in range(3):
        ws = w1[s * EMBED:(s + 1) * EMBED]
        wd.append(jnp.block([[ws, z], [z, ws]]))
    wcat = jnp.concatenate(wd, axis=0)
    w2 = fc2_w.T.astype(jnp.bfloat16)
    w2p = jnp.zeros((2 * HIDDEN, 32), dtype=jnp.bfloat16)
    w2p = w2p.at[:HIDDEN, :NUM_CLASSES].set(w2)
    w2p = w2p.at[HIDDEN:, 16:16 + NUM_CLASSES].set(w2)
    b1d = jnp.concatenate([fc1_b, fc1_b]).reshape(1, 2 * HIDDEN)
    b2p = jnp.zeros((1, 32), dtype=jnp.float32)
    b2p = b2p.at[0, :NUM_CLASSES].set(fc2_b)
    b2p = b2p.at[0, 16:16 + NUM_CLASSES].set(fc2_b)

    g_spec = pl.BlockSpec((BP, 2 * EMBED), lambda i: (i, 0))
    o_spec = pl.BlockSpec((NUM_CLASSES, BP), lambda i: (0, i))
    o_ty = jax.ShapeDtypeStruct((NUM_CLASSES, NPAIR), jnp.float32)
    return pl.pallas_call(
        _mlp_kernel,
        grid=(nb,),
        in_specs=[
            g_spec, g_spec, g_spec,
            pl.BlockSpec((3 * 2 * EMBED, 2 * HIDDEN), lambda i: (0, 0)),
            pl.BlockSpec((1, 2 * HIDDEN), lambda i: (0, 0)),
            pl.BlockSpec((2 * HIDDEN, 32), lambda i: (0, 0)),
            pl.BlockSpec((1, 32), lambda i: (0, 0)),
        ],
        out_specs=(o_spec, o_spec),
        out_shape=(o_ty, o_ty),
    )(gw, gb, gt, wcat, b1d, w2p, b2p)


def kernel(x, emb_word, emb_bigram, emb_trigram, fc1_w, fc1_b, fc2_w, fc2_b):
    xt = jnp.transpose(x, (0, 2, 1))
    iw = xt[0].reshape(N)
    ib = xt[1].reshape(N)
    it = xt[2].reshape(N)
    emb_word, emb_bigram, emb_trigram = (
        lax.optimization_barrier(t.reshape(-1)).reshape(VOCAB, EMBED)
        for t in (emb_word, emb_bigram, emb_trigram))
    panels = []
    for k in range(NSPLIT):
        gw = _sc_gather(iw, emb_word, k * NT)
        gb = _sc_gather(ib, emb_bigram, k * NT)
        gt = _sc_gather(it, emb_trigram, k * NT)
        olo, ohi = _tc_mlp(gw, gb, gt, fc1_w, fc1_b, fc2_w, fc2_b)
        panels += [olo, ohi]
    o_np = jnp.concatenate(panels, axis=1)
    return o_np.reshape(NUM_CLASSES, SEQ, BATCH).transpose(2, 1, 0)
